# baseline (speedup 1.0000x reference)
import jax
import jax.numpy as jnp
from jax import lax
from jax.experimental import pallas as pl
from jax.experimental.pallas import tpu as pltpu
from jax.experimental.pallas import tpu_sc as plsc

W2 = 10
BATCH = 4096
EMB = 64
PACK = 2 * EMB
VOCAB = 1000000
HALF = 524288
N_TOTAL = W2 * BATCH
NC, NS, L = 2, 16, 16
NW = NC * NS
CHUNK = 128
N_PER_W = N_TOTAL // NW
N_CHUNKS = N_PER_W // CHUNK
GROUPS = CHUNK // L
BR = 4096


def _pack_body(x1_ref, x2_ref, o_ref):
    r = lax.broadcasted_iota(jnp.int32, (EMB, EMB), 0)
    c = lax.broadcasted_iota(jnp.int32, (EMB, EMB), 1)
    eye = (r == c).astype(jnp.float32)
    dn = (((0,), (0,)), ((), ()))
    y1 = lax.dot_general(x1_ref[...], eye, dn,
                         preferred_element_type=jnp.float32)
    y2 = lax.dot_general(x2_ref[...], eye, dn,
                         preferred_element_type=jnp.float32)
    o_ref[...] = jnp.concatenate([y1, y2], axis=1)


def _pack_table(table):
    t = table.T
    nblk = HALF // BR
    last = (VOCAB - 1) // BR
    return pl.pallas_call(
        _pack_body,
        grid=(nblk,),
        in_specs=[
            pl.BlockSpec((EMB, BR), lambda i: (0, i)),
            pl.BlockSpec((EMB, BR),
                         lambda i: (0, jnp.minimum(i + nblk, last))),
        ],
        out_specs=pl.BlockSpec((BR, PACK), lambda i: (i, 0)),
        out_shape=jax.ShapeDtypeStruct((HALF, PACK), jnp.float32),
        compiler_params=pltpu.CompilerParams(disable_bounds_checks=True),
    )(t, t)


def _body(cen_ref, ctx_ref, neg_ref, ein_ref, eout_ref,
          pos_out, neg_out,
          idx_c, idx_b, idx_n, pr_c, pr_b, pr_n, bufs, acc_v, sems):
    wid = lax.axis_index("s") * NC + lax.axis_index("c")
    row0 = wid * N_CHUNKS

    lanes = lax.iota(jnp.int32, L)

    def load_idx(j, k):
        pltpu.sync_copy(cen_ref.at[row0 + j], idx_c.at[k])
        pltpu.sync_copy(ctx_ref.at[row0 + j], idx_b.at[k])
        pltpu.sync_copy(neg_ref.at[row0 + j], idx_n.at[k])
        for q in range(CHUNK // L):
            sl = pl.ds(q * L, L)
            for idx, pr in ((idx_c, pr_c), (idx_b, pr_b), (idx_n, pr_n)):
                v = idx[k, sl]
                pr[k, sl] = v - jnp.where(v >= HALF, HALF, 0).astype(jnp.int32)

    def fire(k):
        pltpu.async_copy(ein_ref.at[pr_c.at[k]], bufs.at[k, 0], sems.at[k])
        pltpu.async_copy(eout_ref.at[pr_b.at[k]], bufs.at[k, 1], sems.at[k])
        pltpu.async_copy(eout_ref.at[pr_n.at[k]], bufs.at[k, 2], sems.at[k])

    def drain(k):
        for _ in range(3):
            pltpu.make_async_copy(ein_ref.at[pr_c.at[k]], bufs.at[k, 0],
                                  sems.at[k]).wait()

    def compute(k, acc_p, acc_n):
        buf_a = bufs.at[k, 0]
        buf_b = bufs.at[k, 1]
        buf_n = bufs.at[k, 2]

        def group_step(g, carry):
            acc_p, acc_n = carry
            sl = pl.ds(g * L, L)
            slots = g * L + lanes
            half_c = jnp.where(idx_c[k, sl] >= HALF, EMB, 0).astype(jnp.int32)
            half_b = jnp.where(idx_b[k, sl] >= HALF, EMB, 0).astype(jnp.int32)
            half_n = jnp.where(idx_n[k, sl] >= HALF, EMB, 0).astype(jnp.int32)

            def d_step(ds, carry):
                pp, nn = carry
                for u in range(16):
                    d = ds * 16 + u
                    a = plsc.load_gather(buf_a, [slots, half_c + d])
                    b = plsc.load_gather(buf_b, [slots, half_b + d])
                    c = plsc.load_gather(buf_n, [slots, half_n + d])
                    pp = pp + a * b
                    nn = nn + a * c
                return pp, nn

            zero = jnp.zeros((L,), jnp.float32)
            pred_p, pred_n = lax.fori_loop(0, EMB // 16, d_step, (zero, zero))
            sig_p = 1.0 / (1.0 + jnp.exp(-pred_p))
            sig_n = 1.0 / (1.0 + jnp.exp(-pred_n))
            return acc_p + sig_p, acc_n + sig_n

        return lax.fori_loop(0, GROUPS, group_step, (acc_p, acc_n))

    acc_p = jnp.zeros((L,), jnp.float32)
    acc_n = jnp.zeros((L,), jnp.float32)

    load_idx(0, 0)
    fire(0)
    for j in range(N_CHUNKS):
        k = j % 2
        if j + 1 < N_CHUNKS:
            load_idx(j + 1, 1 - k)
            fire(1 - k)
        drain(k)
        acc_p, acc_n = compute(k, acc_p, acc_n)

    acc_v[...] = acc_p
    pltpu.sync_copy(acc_v, pos_out.at[wid])
    acc_v[...] = acc_n
    pltpu.sync_copy(acc_v, neg_out.at[wid])


@jax.jit
def _w2v_ns_partials(cen, ctx, neg, ein, eout):
    mesh = plsc.VectorSubcoreMesh(core_axis_name="c", subcore_axis_name="s")
    f = pl.kernel(
        _body,
        out_type=(
            jax.ShapeDtypeStruct((NW, L), jnp.float32),
            jax.ShapeDtypeStruct((NW, L), jnp.float32),
        ),
        mesh=mesh,
        scratch_types=[
            pltpu.VMEM((2, CHUNK), jnp.int32),
            pltpu.VMEM((2, CHUNK), jnp.int32),
            pltpu.VMEM((2, CHUNK), jnp.int32),
            pltpu.VMEM((2, CHUNK), jnp.int32),
            pltpu.VMEM((2, CHUNK), jnp.int32),
            pltpu.VMEM((2, CHUNK), jnp.int32),
            pltpu.VMEM((2, 3, CHUNK, PACK), jnp.float32),
            pltpu.VMEM((L,), jnp.float32),
            pltpu.SemaphoreType.DMA((2,)),
        ],
        compiler_params=pltpu.CompilerParams(
            needs_layout_passes=False, disable_bounds_checks=True),
    )
    return f(cen, ctx, neg, ein, eout)


def kernel(center, context, context_negative, emb_in_w, emb_out_w):
    cen = center.reshape(N_TOTAL // CHUNK, CHUNK)
    ctx = context.reshape(N_TOTAL // CHUNK, CHUNK)
    neg = context_negative.reshape(N_TOTAL // CHUNK, CHUNK)
    ein = _pack_table(emb_in_w)
    eout = _pack_table(emb_out_w)
    pos_part, neg_part = _w2v_ns_partials(cen, ctx, neg, ein, eout)
    inv_n = jnp.float32(1.0 / N_TOTAL)
    return 1.0 - jnp.sum(pos_part) * inv_n + jnp.sum(neg_part) * inv_n


# speedup vs baseline: 1.1630x; 1.1630x over previous
import jax
import jax.numpy as jnp
from jax import lax
from jax.experimental import pallas as pl
from jax.experimental.pallas import tpu as pltpu
from jax.experimental.pallas import tpu_sc as plsc

W2 = 10
BATCH = 4096
EMB = 64
PACK = 2 * EMB
VOCAB = 1000000
HALF = 524288
N_TOTAL = W2 * BATCH
NC, NS, L = 2, 16, 16
NW = NC * NS
CHUNK = 128
N_PER_W = N_TOTAL // NW
N_CHUNKS = N_PER_W // CHUNK
GROUPS = CHUNK // L
BR = 4096


def _pack_body(x1_ref, x2_ref, o_ref):
    r = lax.broadcasted_iota(jnp.int32, (EMB, EMB), 0)
    c = lax.broadcasted_iota(jnp.int32, (EMB, EMB), 1)
    eye = (r == c).astype(jnp.float32)
    dn = (((0,), (0,)), ((), ()))
    y1 = lax.dot_general(x1_ref[...], eye, dn,
                         preferred_element_type=jnp.float32)
    y2 = lax.dot_general(x2_ref[...], eye, dn,
                         preferred_element_type=jnp.float32)
    o_ref[...] = jnp.concatenate([y1, y2], axis=1)


def _pack_table(table):
    t = table.T
    nblk = HALF // BR
    last = (VOCAB - 1) // BR
    return pl.pallas_call(
        _pack_body,
        grid=(nblk,),
        in_specs=[
            pl.BlockSpec((EMB, BR), lambda i: (0, i)),
            pl.BlockSpec((EMB, BR),
                         lambda i: (0, jnp.minimum(i + nblk, last))),
        ],
        out_specs=pl.BlockSpec((BR, PACK), lambda i: (i, 0)),
        out_shape=jax.ShapeDtypeStruct((HALF, PACK), jnp.float32),
        compiler_params=pltpu.CompilerParams(disable_bounds_checks=True),
    )(t, t)


def _body(cen_ref, ctx_ref, neg_ref, ein_ref, eout_ref,
          pos_out, neg_out,
          idx_c, idx_b, idx_n, pr_c, pr_b, pr_n, bufs, acc_v, sems):
    wid = lax.axis_index("s") * NC + lax.axis_index("c")
    row0 = wid * N_CHUNKS

    lanes = lax.iota(jnp.int32, L)

    def load_idx(j, k):
        pltpu.sync_copy(cen_ref.at[row0 + j], idx_c.at[k])
        pltpu.sync_copy(ctx_ref.at[row0 + j], idx_b.at[k])
        pltpu.sync_copy(neg_ref.at[row0 + j], idx_n.at[k])
        for q in range(CHUNK // L):
            sl = pl.ds(q * L, L)
            for idx, pr in ((idx_c, pr_c), (idx_b, pr_b), (idx_n, pr_n)):
                v = idx[k, sl]
                pr[k, sl] = v - jnp.where(v >= HALF, HALF, 0).astype(jnp.int32)

    def fire(k):
        pltpu.async_copy(ein_ref.at[pr_c.at[k]], bufs.at[k, 0], sems.at[k])
        pltpu.async_copy(eout_ref.at[pr_b.at[k]], bufs.at[k, 1], sems.at[k])
        pltpu.async_copy(eout_ref.at[pr_n.at[k]], bufs.at[k, 2], sems.at[k])

    def drain(k):
        for _ in range(3):
            pltpu.make_async_copy(ein_ref.at[pr_c.at[k]], bufs.at[k, 0],
                                  sems.at[k]).wait()

    def compute(k, acc_p, acc_n):
        buf_a = bufs.at[k, 0]
        buf_b = bufs.at[k, 1]
        buf_n = bufs.at[k, 2]

        def group_step(g, carry):
            acc_p, acc_n = carry
            sl = pl.ds(g * L, L)
            slots = g * L + lanes
            half_c = jnp.where(idx_c[k, sl] >= HALF, EMB, 0).astype(jnp.int32)
            half_b = jnp.where(idx_b[k, sl] >= HALF, EMB, 0).astype(jnp.int32)
            half_n = jnp.where(idx_n[k, sl] >= HALF, EMB, 0).astype(jnp.int32)

            def d_step(ds, carry):
                pp, nn = carry
                for u in range(16):
                    col = (lanes + (ds * 16 + u)) & (EMB - 1)
                    a = plsc.load_gather(buf_a, [slots, half_c + col])
                    b = plsc.load_gather(buf_b, [slots, half_b + col])
                    c = plsc.load_gather(buf_n, [slots, half_n + col])
                    pp = pp + a * b
                    nn = nn + a * c
                return pp, nn

            zero = jnp.zeros((L,), jnp.float32)
            pred_p, pred_n = lax.fori_loop(0, EMB // 16, d_step, (zero, zero))
            sig_p = 1.0 / (1.0 + jnp.exp(-pred_p))
            sig_n = 1.0 / (1.0 + jnp.exp(-pred_n))
            return acc_p + sig_p, acc_n + sig_n

        return lax.fori_loop(0, GROUPS, group_step, (acc_p, acc_n))

    acc_p = jnp.zeros((L,), jnp.float32)
    acc_n = jnp.zeros((L,), jnp.float32)

    load_idx(0, 0)
    fire(0)
    for j in range(N_CHUNKS):
        k = j % 2
        if j + 1 < N_CHUNKS:
            load_idx(j + 1, 1 - k)
            fire(1 - k)
        drain(k)
        acc_p, acc_n = compute(k, acc_p, acc_n)

    acc_v[...] = acc_p
    pltpu.sync_copy(acc_v, pos_out.at[wid])
    acc_v[...] = acc_n
    pltpu.sync_copy(acc_v, neg_out.at[wid])


@jax.jit
def _w2v_ns_partials(cen, ctx, neg, ein, eout):
    mesh = plsc.VectorSubcoreMesh(core_axis_name="c", subcore_axis_name="s")
    f = pl.kernel(
        _body,
        out_type=(
            jax.ShapeDtypeStruct((NW, L), jnp.float32),
            jax.ShapeDtypeStruct((NW, L), jnp.float32),
        ),
        mesh=mesh,
        scratch_types=[
            pltpu.VMEM((2, CHUNK), jnp.int32),
            pltpu.VMEM((2, CHUNK), jnp.int32),
            pltpu.VMEM((2, CHUNK), jnp.int32),
            pltpu.VMEM((2, CHUNK), jnp.int32),
            pltpu.VMEM((2, CHUNK), jnp.int32),
            pltpu.VMEM((2, CHUNK), jnp.int32),
            pltpu.VMEM((2, 3, CHUNK, PACK), jnp.float32),
            pltpu.VMEM((L,), jnp.float32),
            pltpu.SemaphoreType.DMA((2,)),
        ],
        compiler_params=pltpu.CompilerParams(
            needs_layout_passes=False, disable_bounds_checks=True),
    )
    return f(cen, ctx, neg, ein, eout)


def kernel(center, context, context_negative, emb_in_w, emb_out_w):
    cen = center.reshape(N_TOTAL // CHUNK, CHUNK)
    ctx = context.reshape(N_TOTAL // CHUNK, CHUNK)
    neg = context_negative.reshape(N_TOTAL // CHUNK, CHUNK)
    ein = _pack_table(emb_in_w)
    eout = _pack_table(emb_out_w)
    pos_part, neg_part = _w2v_ns_partials(cen, ctx, neg, ein, eout)
    inv_n = jnp.float32(1.0 / N_TOTAL)
    return 1.0 - jnp.sum(pos_part) * inv_n + jnp.sum(neg_part) * inv_n


# speedup vs baseline: 1.4920x; 1.2828x over previous
import jax
import jax.numpy as jnp
from jax import lax
from jax.experimental import pallas as pl
from jax.experimental.pallas import tpu as pltpu
from jax.experimental.pallas import tpu_sc as plsc

W2 = 10
BATCH = 4096
EMB = 64
PACK = 2 * EMB
VOCAB = 1000000
HALF = 524288
N_TOTAL = W2 * BATCH
NC, NS, L = 2, 16, 16
NW = NC * NS
CHUNK = 128
N_PER_W = N_TOTAL // NW
N_CHUNKS = N_PER_W // CHUNK
GROUPS = CHUNK // L
BR = 4096


def _pack_body(x1_ref, x2_ref, o_ref):
    z = jnp.concatenate([x1_ref[...], x2_ref[...]], axis=0)
    o_ref[...] = z.T


def _pack_table(table):
    t = table.T
    nblk = HALF // BR
    last = (VOCAB - 1) // BR
    return pl.pallas_call(
        _pack_body,
        grid=(nblk,),
        in_specs=[
            pl.BlockSpec((EMB, BR), lambda i: (0, i)),
            pl.BlockSpec((EMB, BR),
                         lambda i: (0, jnp.minimum(i + nblk, last))),
        ],
        out_specs=pl.BlockSpec((BR, PACK), lambda i: (i, 0)),
        out_shape=jax.ShapeDtypeStruct((HALF, PACK), jnp.float32),
        compiler_params=pltpu.CompilerParams(disable_bounds_checks=True),
    )(t, t)


def _body(cen_ref, ctx_ref, neg_ref, ein_ref, eout_ref,
          pos_out, neg_out,
          idx_c, idx_b, idx_n, pr_c, pr_b, pr_n, bufs, acc_v, sems):
    wid = lax.axis_index("s") * NC + lax.axis_index("c")
    row0 = wid * N_CHUNKS

    lanes = lax.iota(jnp.int32, L)

    def load_idx(j, k):
        pltpu.sync_copy(cen_ref.at[row0 + j], idx_c.at[k])
        pltpu.sync_copy(ctx_ref.at[row0 + j], idx_b.at[k])
        pltpu.sync_copy(neg_ref.at[row0 + j], idx_n.at[k])
        for q in range(CHUNK // L):
            sl = pl.ds(q * L, L)
            for idx, pr in ((idx_c, pr_c), (idx_b, pr_b), (idx_n, pr_n)):
                v = idx[k, sl]
                pr[k, sl] = v - jnp.where(v >= HALF, HALF, 0).astype(jnp.int32)

    def fire(k):
        pltpu.async_copy(ein_ref.at[pr_c.at[k]], bufs.at[k, 0], sems.at[k])
        pltpu.async_copy(eout_ref.at[pr_b.at[k]], bufs.at[k, 1], sems.at[k])
        pltpu.async_copy(eout_ref.at[pr_n.at[k]], bufs.at[k, 2], sems.at[k])

    def drain(k):
        for _ in range(3):
            pltpu.make_async_copy(ein_ref.at[pr_c.at[k]], bufs.at[k, 0],
                                  sems.at[k]).wait()

    def compute(k, acc_p, acc_n):
        buf_a = bufs.at[k, 0]
        buf_b = bufs.at[k, 1]
        buf_n = bufs.at[k, 2]

        def group_step(g, carry):
            acc_p, acc_n = carry
            sl = pl.ds(g * L, L)
            slots = g * L + lanes
            half_c = jnp.where(idx_c[k, sl] >= HALF, EMB, 0).astype(jnp.int32)
            half_b = jnp.where(idx_b[k, sl] >= HALF, EMB, 0).astype(jnp.int32)
            half_n = jnp.where(idx_n[k, sl] >= HALF, EMB, 0).astype(jnp.int32)

            def d_step(ds, carry):
                pp, nn = carry
                for u in range(16):
                    col = (lanes + (ds * 16 + u)) & (EMB - 1)
                    a = plsc.load_gather(buf_a, [slots, half_c + col])
                    b = plsc.load_gather(buf_b, [slots, half_b + col])
                    c = plsc.load_gather(buf_n, [slots, half_n + col])
                    pp = pp + a * b
                    nn = nn + a * c
                return pp, nn

            zero = jnp.zeros((L,), jnp.float32)
            pred_p, pred_n = lax.fori_loop(0, EMB // 16, d_step, (zero, zero))
            sig_p = 1.0 / (1.0 + jnp.exp(-pred_p))
            sig_n = 1.0 / (1.0 + jnp.exp(-pred_n))
            return acc_p + sig_p, acc_n + sig_n

        return lax.fori_loop(0, GROUPS, group_step, (acc_p, acc_n))

    acc_p = jnp.zeros((L,), jnp.float32)
    acc_n = jnp.zeros((L,), jnp.float32)

    load_idx(0, 0)
    fire(0)
    for j in range(N_CHUNKS):
        k = j % 2
        if j + 1 < N_CHUNKS:
            load_idx(j + 1, 1 - k)
            fire(1 - k)
        drain(k)
        acc_p, acc_n = compute(k, acc_p, acc_n)

    acc_v[...] = acc_p
    pltpu.sync_copy(acc_v, pos_out.at[wid])
    acc_v[...] = acc_n
    pltpu.sync_copy(acc_v, neg_out.at[wid])


@jax.jit
def _w2v_ns_partials(cen, ctx, neg, ein, eout):
    mesh = plsc.VectorSubcoreMesh(core_axis_name="c", subcore_axis_name="s")
    f = pl.kernel(
        _body,
        out_type=(
            jax.ShapeDtypeStruct((NW, L), jnp.float32),
            jax.ShapeDtypeStruct((NW, L), jnp.float32),
        ),
        mesh=mesh,
        scratch_types=[
            pltpu.VMEM((2, CHUNK), jnp.int32),
            pltpu.VMEM((2, CHUNK), jnp.int32),
            pltpu.VMEM((2, CHUNK), jnp.int32),
            pltpu.VMEM((2, CHUNK), jnp.int32),
            pltpu.VMEM((2, CHUNK), jnp.int32),
            pltpu.VMEM((2, CHUNK), jnp.int32),
            pltpu.VMEM((2, 3, CHUNK, PACK), jnp.float32),
            pltpu.VMEM((L,), jnp.float32),
            pltpu.SemaphoreType.DMA((2,)),
        ],
        compiler_params=pltpu.CompilerParams(
            needs_layout_passes=False, disable_bounds_checks=True),
    )
    return f(cen, ctx, neg, ein, eout)


def kernel(center, context, context_negative, emb_in_w, emb_out_w):
    cen = center.reshape(N_TOTAL // CHUNK, CHUNK)
    ctx = context.reshape(N_TOTAL // CHUNK, CHUNK)
    neg = context_negative.reshape(N_TOTAL // CHUNK, CHUNK)
    ein = _pack_table(emb_in_w)
    eout = _pack_table(emb_out_w)
    pos_part, neg_part = _w2v_ns_partials(cen, ctx, neg, ein, eout)
    inv_n = jnp.float32(1.0 / N_TOTAL)
    return 1.0 - jnp.sum(pos_part) * inv_n + jnp.sum(neg_part) * inv_n


# speedup vs baseline: 1.7193x; 1.1524x over previous
import jax
import jax.numpy as jnp
from jax import lax
from jax.experimental import pallas as pl
from jax.experimental.pallas import tpu as pltpu
from jax.experimental.pallas import tpu_sc as plsc

W2 = 10
BATCH = 4096
EMB = 64
PACK = 2 * EMB
VOCAB = 1000000
HALF = 524288
N_TOTAL = W2 * BATCH
NC, NS, L = 2, 16, 16
NW = NC * NS
CHUNK = 128
N_PER_W = N_TOTAL // NW
N_CHUNKS = N_PER_W // CHUNK
GROUPS = CHUNK // L
BR = 8192


def _pack_body(x1_ref, x2_ref, o_ref):
    z = jnp.concatenate([x1_ref[...], x2_ref[...]], axis=0)
    o_ref[...] = z.T


def _pack_table(table):
    t = table.T
    nblk = HALF // BR
    last = (VOCAB - 1) // BR
    return pl.pallas_call(
        _pack_body,
        grid=(nblk,),
        in_specs=[
            pl.BlockSpec((EMB, BR), lambda i: (0, i)),
            pl.BlockSpec((EMB, BR),
                         lambda i: (0, jnp.minimum(i + nblk, last))),
        ],
        out_specs=pl.BlockSpec((BR, PACK), lambda i: (i, 0)),
        out_shape=jax.ShapeDtypeStruct((HALF, PACK), jnp.float32),
        compiler_params=pltpu.CompilerParams(disable_bounds_checks=True),
    )(t, t)


def _body(cen_ref, ctx_ref, neg_ref, ein_ref, eout_ref,
          pos_out, neg_out,
          idx_c, idx_b, idx_n, pr_c, pr_b, pr_n, bufs, acc_v, sems):
    wid = lax.axis_index("s") * NC + lax.axis_index("c")
    row0 = wid * N_CHUNKS

    lanes = lax.iota(jnp.int32, L)

    def load_idx(j, k):
        pltpu.sync_copy(cen_ref.at[row0 + j], idx_c.at[k])
        pltpu.sync_copy(ctx_ref.at[row0 + j], idx_b.at[k])
        pltpu.sync_copy(neg_ref.at[row0 + j], idx_n.at[k])
        for q in range(CHUNK // L):
            sl = pl.ds(q * L, L)
            for idx, pr in ((idx_c, pr_c), (idx_b, pr_b), (idx_n, pr_n)):
                v = idx[k, sl]
                pr[k, sl] = v - jnp.where(v >= HALF, HALF, 0).astype(jnp.int32)

    def fire(k):
        pltpu.async_copy(ein_ref.at[pr_c.at[k]], bufs.at[k, 0], sems.at[k])
        pltpu.async_copy(eout_ref.at[pr_b.at[k]], bufs.at[k, 1], sems.at[k])
        pltpu.async_copy(eout_ref.at[pr_n.at[k]], bufs.at[k, 2], sems.at[k])

    def drain(k):
        for _ in range(3):
            pltpu.make_async_copy(ein_ref.at[pr_c.at[k]], bufs.at[k, 0],
                                  sems.at[k]).wait()

    def compute(k, acc_p, acc_n):
        buf_a = bufs.at[k, 0]
        buf_b = bufs.at[k, 1]
        buf_n = bufs.at[k, 2]

        def group_step(g, carry):
            acc_p, acc_n = carry
            sl = pl.ds(g * L, L)
            slots = g * L + lanes
            half_c = jnp.where(idx_c[k, sl] >= HALF, EMB, 0).astype(jnp.int32)
            half_b = jnp.where(idx_b[k, sl] >= HALF, EMB, 0).astype(jnp.int32)
            half_n = jnp.where(idx_n[k, sl] >= HALF, EMB, 0).astype(jnp.int32)

            def d_step(ds, carry):
                pp, nn = carry
                for u in range(16):
                    col = (lanes + (ds * 16 + u)) & (EMB - 1)
                    a = plsc.load_gather(buf_a, [slots, half_c + col])
                    b = plsc.load_gather(buf_b, [slots, half_b + col])
                    c = plsc.load_gather(buf_n, [slots, half_n + col])
                    pp = pp + a * b
                    nn = nn + a * c
                return pp, nn

            zero = jnp.zeros((L,), jnp.float32)
            pred_p, pred_n = lax.fori_loop(0, EMB // 16, d_step, (zero, zero))
            sig_p = 1.0 / (1.0 + jnp.exp(-pred_p))
            sig_n = 1.0 / (1.0 + jnp.exp(-pred_n))
            return acc_p + sig_p, acc_n + sig_n

        return lax.fori_loop(0, GROUPS, group_step, (acc_p, acc_n))

    acc_p = jnp.zeros((L,), jnp.float32)
    acc_n = jnp.zeros((L,), jnp.float32)

    load_idx(0, 0)
    fire(0)
    for j in range(N_CHUNKS):
        k = j % 2
        if j + 1 < N_CHUNKS:
            load_idx(j + 1, 1 - k)
            fire(1 - k)
        drain(k)
        acc_p, acc_n = compute(k, acc_p, acc_n)

    acc_v[...] = acc_p
    pltpu.sync_copy(acc_v, pos_out.at[wid])
    acc_v[...] = acc_n
    pltpu.sync_copy(acc_v, neg_out.at[wid])


@jax.jit
def _w2v_ns_partials(cen, ctx, neg, ein, eout):
    mesh = plsc.VectorSubcoreMesh(core_axis_name="c", subcore_axis_name="s")
    f = pl.kernel(
        _body,
        out_type=(
            jax.ShapeDtypeStruct((NW, L), jnp.float32),
            jax.ShapeDtypeStruct((NW, L), jnp.float32),
        ),
        mesh=mesh,
        scratch_types=[
            pltpu.VMEM((2, CHUNK), jnp.int32),
            pltpu.VMEM((2, CHUNK), jnp.int32),
            pltpu.VMEM((2, CHUNK), jnp.int32),
            pltpu.VMEM((2, CHUNK), jnp.int32),
            pltpu.VMEM((2, CHUNK), jnp.int32),
            pltpu.VMEM((2, CHUNK), jnp.int32),
            pltpu.VMEM((2, 3, CHUNK, PACK), jnp.float32),
            pltpu.VMEM((L,), jnp.float32),
            pltpu.SemaphoreType.DMA((2,)),
        ],
        compiler_params=pltpu.CompilerParams(
            needs_layout_passes=False, disable_bounds_checks=True),
    )
    return f(cen, ctx, neg, ein, eout)


def kernel(center, context, context_negative, emb_in_w, emb_out_w):
    cen = center.reshape(N_TOTAL // CHUNK, CHUNK)
    ctx = context.reshape(N_TOTAL // CHUNK, CHUNK)
    neg = context_negative.reshape(N_TOTAL // CHUNK, CHUNK)
    ein = _pack_table(emb_in_w)
    eout = _pack_table(emb_out_w)
    pos_part, neg_part = _w2v_ns_partials(cen, ctx, neg, ein, eout)
    inv_n = jnp.float32(1.0 / N_TOTAL)
    return 1.0 - jnp.sum(pos_part) * inv_n + jnp.sum(neg_part) * inv_n


# speedup vs baseline: 1.7657x; 1.0270x over previous
import jax
import jax.numpy as jnp
from jax import lax
from jax.experimental import pallas as pl
from jax.experimental.pallas import tpu as pltpu
from jax.experimental.pallas import tpu_sc as plsc

W2 = 10
BATCH = 4096
EMB = 64
PACK = 2 * EMB
VOCAB = 1000000
HALF = 524288
N_TOTAL = W2 * BATCH
NC, NS, L = 2, 16, 16
NW = NC * NS
CHUNK = 128
N_PER_W = N_TOTAL // NW
N_CHUNKS = N_PER_W // CHUNK
GROUPS = CHUNK // L
BR = 16384


def _pack_body(x1_ref, x2_ref, o_ref):
    z = jnp.concatenate([x1_ref[...], x2_ref[...]], axis=0)
    o_ref[...] = z.T


def _pack_table(table):
    t = table.T
    nblk = HALF // BR
    last = (VOCAB - 1) // BR
    return pl.pallas_call(
        _pack_body,
        grid=(nblk,),
        in_specs=[
            pl.BlockSpec((EMB, BR), lambda i: (0, i)),
            pl.BlockSpec((EMB, BR),
                         lambda i: (0, jnp.minimum(i + nblk, last))),
        ],
        out_specs=pl.BlockSpec((BR, PACK), lambda i: (i, 0)),
        out_shape=jax.ShapeDtypeStruct((HALF, PACK), jnp.float32),
        compiler_params=pltpu.CompilerParams(disable_bounds_checks=True),
    )(t, t)


def _body(cen_ref, ctx_ref, neg_ref, ein_ref, eout_ref,
          pos_out, neg_out,
          idx_c, idx_b, idx_n, pr_c, pr_b, pr_n, bufs, acc_v, sems):
    wid = lax.axis_index("s") * NC + lax.axis_index("c")
    row0 = wid * N_CHUNKS

    lanes = lax.iota(jnp.int32, L)

    def load_idx(j, k):
        pltpu.sync_copy(cen_ref.at[row0 + j], idx_c.at[k])
        pltpu.sync_copy(ctx_ref.at[row0 + j], idx_b.at[k])
        pltpu.sync_copy(neg_ref.at[row0 + j], idx_n.at[k])
        for q in range(CHUNK // L):
            sl = pl.ds(q * L, L)
            for idx, pr in ((idx_c, pr_c), (idx_b, pr_b), (idx_n, pr_n)):
                v = idx[k, sl]
                pr[k, sl] = v - jnp.where(v >= HALF, HALF, 0).astype(jnp.int32)

    def fire(k):
        pltpu.async_copy(ein_ref.at[pr_c.at[k]], bufs.at[k, 0], sems.at[k])
        pltpu.async_copy(eout_ref.at[pr_b.at[k]], bufs.at[k, 1], sems.at[k])
        pltpu.async_copy(eout_ref.at[pr_n.at[k]], bufs.at[k, 2], sems.at[k])

    def drain(k):
        for _ in range(3):
            pltpu.make_async_copy(ein_ref.at[pr_c.at[k]], bufs.at[k, 0],
                                  sems.at[k]).wait()

    def compute(k, acc_p, acc_n):
        buf_a = bufs.at[k, 0]
        buf_b = bufs.at[k, 1]
        buf_n = bufs.at[k, 2]

        def group_step(g, carry):
            acc_p, acc_n = carry
            sl = pl.ds(g * L, L)
            slots = g * L + lanes
            half_c = jnp.where(idx_c[k, sl] >= HALF, EMB, 0).astype(jnp.int32)
            half_b = jnp.where(idx_b[k, sl] >= HALF, EMB, 0).astype(jnp.int32)
            half_n = jnp.where(idx_n[k, sl] >= HALF, EMB, 0).astype(jnp.int32)

            def d_step(ds, carry):
                pp, nn = carry
                for u in range(16):
                    col = (lanes + (ds * 16 + u)) & (EMB - 1)
                    a = plsc.load_gather(buf_a, [slots, half_c + col])
                    b = plsc.load_gather(buf_b, [slots, half_b + col])
                    c = plsc.load_gather(buf_n, [slots, half_n + col])
                    pp = pp + a * b
                    nn = nn + a * c
                return pp, nn

            zero = jnp.zeros((L,), jnp.float32)
            pred_p, pred_n = lax.fori_loop(0, EMB // 16, d_step, (zero, zero))
            sig_p = 1.0 / (1.0 + jnp.exp(-pred_p))
            sig_n = 1.0 / (1.0 + jnp.exp(-pred_n))
            return acc_p + sig_p, acc_n + sig_n

        return lax.fori_loop(0, GROUPS, group_step, (acc_p, acc_n))

    acc_p = jnp.zeros((L,), jnp.float32)
    acc_n = jnp.zeros((L,), jnp.float32)

    load_idx(0, 0)
    fire(0)
    for j in range(N_CHUNKS):
        k = j % 2
        if j + 1 < N_CHUNKS:
            load_idx(j + 1, 1 - k)
            fire(1 - k)
        drain(k)
        acc_p, acc_n = compute(k, acc_p, acc_n)

    acc_v[...] = acc_p
    pltpu.sync_copy(acc_v, pos_out.at[wid])
    acc_v[...] = acc_n
    pltpu.sync_copy(acc_v, neg_out.at[wid])


@jax.jit
def _w2v_ns_partials(cen, ctx, neg, ein, eout):
    mesh = plsc.VectorSubcoreMesh(core_axis_name="c", subcore_axis_name="s")
    f = pl.kernel(
        _body,
        out_type=(
            jax.ShapeDtypeStruct((NW, L), jnp.float32),
            jax.ShapeDtypeStruct((NW, L), jnp.float32),
        ),
        mesh=mesh,
        scratch_types=[
            pltpu.VMEM((2, CHUNK), jnp.int32),
            pltpu.VMEM((2, CHUNK), jnp.int32),
            pltpu.VMEM((2, CHUNK), jnp.int32),
            pltpu.VMEM((2, CHUNK), jnp.int32),
            pltpu.VMEM((2, CHUNK), jnp.int32),
            pltpu.VMEM((2, CHUNK), jnp.int32),
            pltpu.VMEM((2, 3, CHUNK, PACK), jnp.float32),
            pltpu.VMEM((L,), jnp.float32),
            pltpu.SemaphoreType.DMA((2,)),
        ],
        compiler_params=pltpu.CompilerParams(
            needs_layout_passes=False, disable_bounds_checks=True),
    )
    return f(cen, ctx, neg, ein, eout)


def kernel(center, context, context_negative, emb_in_w, emb_out_w):
    cen = center.reshape(N_TOTAL // CHUNK, CHUNK)
    ctx = context.reshape(N_TOTAL // CHUNK, CHUNK)
    neg = context_negative.reshape(N_TOTAL // CHUNK, CHUNK)
    ein = _pack_table(emb_in_w)
    eout = _pack_table(emb_out_w)
    pos_part, neg_part = _w2v_ns_partials(cen, ctx, neg, ein, eout)
    inv_n = jnp.float32(1.0 / N_TOTAL)
    return 1.0 - jnp.sum(pos_part) * inv_n + jnp.sum(neg_part) * inv_n


# speedup vs baseline: 1.7948x; 1.0165x over previous
import jax
import jax.numpy as jnp
from jax import lax
from jax.experimental import pallas as pl
from jax.experimental.pallas import tpu as pltpu
from jax.experimental.pallas import tpu_sc as plsc

W2 = 10
BATCH = 4096
EMB = 64
PACK = 2 * EMB
VOCAB = 1000000
HALF = 524288
N_TOTAL = W2 * BATCH
NC, NS, L = 2, 16, 16
NW = NC * NS
CHUNK = 128
N_PER_W = N_TOTAL // NW
N_CHUNKS = N_PER_W // CHUNK
GROUPS = CHUNK // L
BR = 16384


def _pack_body(x1_ref, x2_ref, o_ref):
    z = jnp.concatenate([x1_ref[...], x2_ref[...]], axis=0)
    o_ref[...] = z.T


def _pack_table(table):
    t = table.T
    nblk = HALF // BR
    last = (VOCAB - 1) // BR
    return pl.pallas_call(
        _pack_body,
        grid=(nblk,),
        in_specs=[
            pl.BlockSpec((EMB, BR), lambda i: (0, i)),
            pl.BlockSpec((EMB, BR),
                         lambda i: (0, jnp.minimum(i + nblk, last))),
        ],
        out_specs=pl.BlockSpec((BR, PACK), lambda i: (i, 0)),
        out_shape=jax.ShapeDtypeStruct((HALF, PACK), jnp.float32),
        compiler_params=pltpu.CompilerParams(disable_bounds_checks=True),
    )(t, t)


def _body(cen_ref, ctx_ref, neg_ref, ein_ref, eout_ref,
          pos_out, neg_out,
          idx_c, idx_b, idx_n, pr_c, pr_b, pr_n, bufs, acc_v, sems):
    wid = lax.axis_index("s") * NC + lax.axis_index("c")
    row0 = wid * N_CHUNKS

    lanes = lax.iota(jnp.int32, L)

    def load_idx(j, k):
        pltpu.sync_copy(cen_ref.at[row0 + j], idx_c.at[k])
        pltpu.sync_copy(ctx_ref.at[row0 + j], idx_b.at[k])
        pltpu.sync_copy(neg_ref.at[row0 + j], idx_n.at[k])
        for q in range(CHUNK // L):
            sl = pl.ds(q * L, L)
            for idx, pr in ((idx_c, pr_c), (idx_b, pr_b), (idx_n, pr_n)):
                v = idx[k, sl]
                hi = jnp.where(v >= HALF, 1, 0).astype(jnp.int32)
                pr[k, sl] = 2 * (v - hi * HALF) + hi

    def fire(k):
        pltpu.async_copy(ein_ref.at[pr_c.at[k]], bufs.at[k, 0], sems.at[k])
        pltpu.async_copy(eout_ref.at[pr_b.at[k]], bufs.at[k, 1], sems.at[k])
        pltpu.async_copy(eout_ref.at[pr_n.at[k]], bufs.at[k, 2], sems.at[k])

    def drain(k):
        for _ in range(3):
            pltpu.make_async_copy(ein_ref.at[pr_c.at[k]], bufs.at[k, 0],
                                  sems.at[k]).wait()

    def compute(k, acc_p, acc_n):
        buf_a = bufs.at[k, 0]
        buf_b = bufs.at[k, 1]
        buf_n = bufs.at[k, 2]

        def group_step(g, carry):
            acc_p, acc_n = carry
            slots = g * L + lanes

            def d_step(ds, carry):
                pp, nn = carry
                for u in range(16):
                    col = (lanes + (ds * 16 + u)) & (EMB - 1)
                    a = plsc.load_gather(buf_a, [slots, col])
                    b = plsc.load_gather(buf_b, [slots, col])
                    c = plsc.load_gather(buf_n, [slots, col])
                    pp = pp + a * b
                    nn = nn + a * c
                return pp, nn

            zero = jnp.zeros((L,), jnp.float32)
            pred_p, pred_n = lax.fori_loop(0, EMB // 16, d_step, (zero, zero))
            sig_p = 1.0 / (1.0 + jnp.exp(-pred_p))
            sig_n = 1.0 / (1.0 + jnp.exp(-pred_n))
            return acc_p + sig_p, acc_n + sig_n

        return lax.fori_loop(0, GROUPS, group_step, (acc_p, acc_n))

    acc_p = jnp.zeros((L,), jnp.float32)
    acc_n = jnp.zeros((L,), jnp.float32)

    load_idx(0, 0)
    fire(0)
    for j in range(N_CHUNKS):
        k = j % 2
        if j + 1 < N_CHUNKS:
            load_idx(j + 1, 1 - k)
            fire(1 - k)
        drain(k)
        acc_p, acc_n = compute(k, acc_p, acc_n)

    acc_v[...] = acc_p
    pltpu.sync_copy(acc_v, pos_out.at[wid])
    acc_v[...] = acc_n
    pltpu.sync_copy(acc_v, neg_out.at[wid])


@jax.jit
def _w2v_ns_partials(cen, ctx, neg, ein, eout):
    mesh = plsc.VectorSubcoreMesh(core_axis_name="c", subcore_axis_name="s")
    f = pl.kernel(
        _body,
        out_type=(
            jax.ShapeDtypeStruct((NW, L), jnp.float32),
            jax.ShapeDtypeStruct((NW, L), jnp.float32),
        ),
        mesh=mesh,
        scratch_types=[
            pltpu.VMEM((2, CHUNK), jnp.int32),
            pltpu.VMEM((2, CHUNK), jnp.int32),
            pltpu.VMEM((2, CHUNK), jnp.int32),
            pltpu.VMEM((2, CHUNK), jnp.int32),
            pltpu.VMEM((2, CHUNK), jnp.int32),
            pltpu.VMEM((2, CHUNK), jnp.int32),
            pltpu.VMEM((2, 3, CHUNK, EMB), jnp.float32),
            pltpu.VMEM((L,), jnp.float32),
            pltpu.SemaphoreType.DMA((2,)),
        ],
        compiler_params=pltpu.CompilerParams(
            needs_layout_passes=False, disable_bounds_checks=True,
            use_tc_tiling_on_sc=False),
    )
    return f(cen, ctx, neg, ein, eout)


def kernel(center, context, context_negative, emb_in_w, emb_out_w):
    cen = center.reshape(N_TOTAL // CHUNK, CHUNK)
    ctx = context.reshape(N_TOTAL // CHUNK, CHUNK)
    neg = context_negative.reshape(N_TOTAL // CHUNK, CHUNK)
    ein = _pack_table(emb_in_w).reshape(2 * HALF, EMB)
    eout = _pack_table(emb_out_w).reshape(2 * HALF, EMB)
    pos_part, neg_part = _w2v_ns_partials(cen, ctx, neg, ein, eout)
    inv_n = jnp.float32(1.0 / N_TOTAL)
    return 1.0 - jnp.sum(pos_part) * inv_n + jnp.sum(neg_part) * inv_n
